# Initial kernel scaffold; baseline (speedup 1.0000x reference)
import jax
import jax.numpy as jnp
from jax.experimental import pallas as pl
from jax.experimental.pallas import tpu as pltpu

_BLK_S = 512


def _add_body(x_ref, pos_ref, o_ref):
    o_ref[...] = x_ref[...] + pos_ref[None, :, :]


def kernel(x, pos_embedding):
    batch, seq, hid = x.shape
    grid = (seq // _BLK_S, batch)
    return pl.pallas_call(
        _add_body,
        grid=grid,
        in_specs=[
            pl.BlockSpec((1, _BLK_S, hid), lambda s, b: (b, s, 0)),
            pl.BlockSpec((_BLK_S, hid), lambda s, b: (s, 0)),
        ],
        out_specs=pl.BlockSpec((1, _BLK_S, hid), lambda s, b: (b, s, 0)),
        out_shape=jax.ShapeDtypeStruct(x.shape, x.dtype),
        compiler_params=pltpu.CompilerParams(
            dimension_semantics=("arbitrary", "arbitrary"),
        ),
    )(x, pos_embedding)


# speedup vs baseline: 2.8362x; 2.8362x over previous
import jax
import jax.numpy as jnp
from jax.experimental import pallas as pl
from jax.experimental.pallas import tpu as pltpu

_BLK_S = 512


def _add_body(x_ref, pos_ref, o_ref):
    o_ref[...] = x_ref[...] + pos_ref[...][None, :, :]


def kernel(x, pos_embedding):
    batch, seq, hid = x.shape
    grid = (seq // _BLK_S, batch)
    return pl.pallas_call(
        _add_body,
        grid=grid,
        in_specs=[
            pl.BlockSpec((1, _BLK_S, hid), lambda s, b: (b, s, 0)),
            pl.BlockSpec((_BLK_S, hid), lambda s, b: (s, 0)),
        ],
        out_specs=pl.BlockSpec((1, _BLK_S, hid), lambda s, b: (b, s, 0)),
        out_shape=jax.ShapeDtypeStruct(x.shape, x.dtype),
        compiler_params=pltpu.CompilerParams(
            dimension_semantics=("arbitrary", "arbitrary"),
        ),
    )(x, pos_embedding)


# speedup vs baseline: 3.1708x; 1.1180x over previous
import jax
import jax.numpy as jnp
from jax.experimental import pallas as pl
from jax.experimental.pallas import tpu as pltpu

_BLK_S = 1024


def _add_body(x_ref, pos_ref, o_ref):
    o_ref[...] = x_ref[...] + pos_ref[...][None, :, :]


def kernel(x, pos_embedding):
    batch, seq, hid = x.shape
    grid = (seq // _BLK_S, batch)
    return pl.pallas_call(
        _add_body,
        grid=grid,
        in_specs=[
            pl.BlockSpec((1, _BLK_S, hid), lambda s, b: (b, s, 0)),
            pl.BlockSpec((_BLK_S, hid), lambda s, b: (s, 0)),
        ],
        out_specs=pl.BlockSpec((1, _BLK_S, hid), lambda s, b: (b, s, 0)),
        out_shape=jax.ShapeDtypeStruct(x.shape, x.dtype),
        compiler_params=pltpu.CompilerParams(
            dimension_semantics=("arbitrary", "arbitrary"),
        ),
    )(x, pos_embedding)


# speedup vs baseline: 3.3026x; 1.0416x over previous
import jax
import jax.numpy as jnp
from jax.experimental import pallas as pl
from jax.experimental.pallas import tpu as pltpu

_BLK_S = 2048


def _add_body(x_ref, pos_ref, o_ref):
    o_ref[...] = x_ref[...] + pos_ref[...][None, :, :]


def kernel(x, pos_embedding):
    batch, seq, hid = x.shape
    grid = (seq // _BLK_S, batch)
    return pl.pallas_call(
        _add_body,
        grid=grid,
        in_specs=[
            pl.BlockSpec((1, _BLK_S, hid), lambda s, b: (b, s, 0)),
            pl.BlockSpec((_BLK_S, hid), lambda s, b: (s, 0)),
        ],
        out_specs=pl.BlockSpec((1, _BLK_S, hid), lambda s, b: (b, s, 0)),
        out_shape=jax.ShapeDtypeStruct(x.shape, x.dtype),
        compiler_params=pltpu.CompilerParams(
            dimension_semantics=("arbitrary", "arbitrary"),
        ),
    )(x, pos_embedding)
